# baseline (speedup 1.0000x reference)
import dataclasses
import functools

import jax
import jax.numpy as jnp
from jax import lax
from jax.experimental import pallas as pl
from jax.experimental.pallas import tpu as pltpu
from jax.experimental.pallas import tpu_sc as plsc

N = 1024
E = 16384
HID = 128
DE = 4
NH = 16
ALPHA = 0.2
NEG = -1e9
O1 = 2 * HID
CAT = NH * O1
NSUB = 16

_SC_PARAMS = pltpu.CompilerParams()
if "needs_layout_passes" in pltpu.CompilerParams.__dataclass_fields__:
    _SC_PARAMS = dataclasses.replace(_SC_PARAMS, needs_layout_passes=False)


def _softcol(s):
    m = jnp.max(s, axis=0, keepdims=True)
    e = jnp.exp(s - m)
    return e * (1.0 / jnp.sum(e, axis=0, keepdims=True))


def _elu(x):
    return jnp.where(x > 0, x, jnp.exp(jnp.minimum(x, 0.0)) - 1.0)


def _p12_body(x_ref, whw_ref, g1w_ref, g1b_ref, wgcat_ref, atop_ref,
              abot_ref, ea_ref, wegf_ref, ae2_ref,
              gp1_ref, whcat_ref, f1_ref, f2t_ref, be_ref):
    h = jnp.dot(x_ref[...], whw_ref[...], preferred_element_type=jnp.float32)
    h = _elu(h)
    s = jax.nn.sigmoid(jnp.dot(h, g1w_ref[...]) + g1b_ref[0, 0])
    g = _softcol(s)
    gp1_ref[...] = jnp.sum(g * h, axis=0, keepdims=True)
    whcat = jnp.dot(h, wgcat_ref[...], preferred_element_type=jnp.float32)
    whcat_ref[...] = whcat
    for hd in range(NH):
        wh = whcat[:, O1 * hd:O1 * (hd + 1)]
        f1_ref[:, hd:hd + 1] = lax.dot_general(
            wh, atop_ref[hd:hd + 1, :], (((1,), (1,)), ((), ())))
        f2t_ref[hd:hd + 1, :] = lax.dot_general(
            abot_ref[hd:hd + 1, :], wh, (((1,), (1,)), ((), ())))
    wbe = jnp.dot(wegf_ref[...], ae2_ref[...])
    be_ref[...] = jnp.dot(ea_ref[...], wbe)


def _a1_body(whcat_ref, f1_ref, f2t_ref, aeg_ref, adj_ref,
             nn_ref, u1_ref, v1_ref, bp1_ref,
             hcat_ref, p1_ref, p2_ref):
    adjm = adj_ref[...] > 0
    nk = tuple(nn_ref[k] for k in range(DE))
    bs = f1_ref.shape[0]
    p1 = jnp.zeros((bs, 1), jnp.float32) + bp1_ref[0, 0]
    p2 = jnp.zeros((bs, 1), jnp.float32)
    for hd in range(NH):
        wh = whcat_ref[:, O1 * hd:O1 * (hd + 1)]
        bt = (nk[0] * aeg_ref[hd, 0] + nk[1] * aeg_ref[hd, 1]
              + nk[2] * aeg_ref[hd, 2] + nk[3] * aeg_ref[hd, 3])
        t = f1_ref[:, hd:hd + 1] + f2t_ref[hd:hd + 1, :] + bt
        t = jnp.where(t > 0, t, ALPHA * t)
        e = jnp.where(adjm, t, NEG)
        mx = jnp.max(e, axis=1, keepdims=True)
        ex = jnp.exp(e - mx)
        att = ex * (1.0 / jnp.sum(ex, axis=1, keepdims=True))
        hp = jnp.dot(att, wh, preferred_element_type=jnp.float32)
        hp = _elu(hp)
        hcat_ref[:, O1 * hd:O1 * (hd + 1)] = hp
        p1 = p1 + jnp.dot(hp, u1_ref[O1 * hd:O1 * (hd + 1), :])
        p2 = p2 + jnp.dot(hp, v1_ref[O1 * hd:O1 * (hd + 1), :])
    p1_ref[...] = p1
    p2_ref[...] = p2


def _pa_body(hcat_ref, gate_ref, pm_ref, g2w_ref, g2b_ref, w2_ref, a2t_ref,
             a2b_ref, b2_ref, mf_ref, u2_ref, v2_ref, bp2_ref,
             gp2_ref, h2_ref, p1b_ref, p2b_ref):
    nh = jnp.dot(pm_ref[...], hcat_ref[...] * gate_ref[...],
                 preferred_element_type=jnp.float32)
    s = jax.nn.sigmoid(jnp.dot(nh, g2w_ref[...]) + g2b_ref[0, 0])
    g = _softcol(s)
    gp2_ref[...] = jnp.sum(g * nh, axis=0, keepdims=True)
    wh2 = jnp.dot(nh, w2_ref[...], preferred_element_type=jnp.float32)
    f12 = lax.dot_general(wh2, a2t_ref[...], (((1,), (1,)), ((), ())))
    f2t2 = lax.dot_general(a2b_ref[...], wh2, (((1,), (1,)), ((), ())))
    t = f12 + f2t2 + b2_ref[...]
    t = jnp.where(t > 0, t, ALPHA * t)
    e = jnp.where(mf_ref[...] > 0, t, NEG)
    mx = jnp.max(e, axis=1, keepdims=True)
    ex = jnp.exp(e - mx)
    att = ex * (1.0 / jnp.sum(ex, axis=1, keepdims=True))
    h2 = jnp.dot(att, wh2, preferred_element_type=jnp.float32)
    h2_ref[...] = h2
    p1b_ref[...] = jnp.dot(h2, u2_ref[...]) + bp2_ref[0, 0]
    p2b_ref[...] = jnp.dot(h2, v2_ref[...])


def _f_body(h2r1_ref, g21_ref, h2r2_ref, g22_ref, pm2_ref, g3w_ref, g3b_ref,
            gp11_ref, gp21_ref, gp12_ref, gp22_ref,
            wx0_ref, wh0_ref, b0_ref, wx1_ref, wh1_ref, b1_ref,
            wfc_ref, bfc_ref, out_ref):
    def gvec(h2r_ref, g2_ref, gp1_ref, gp2_ref):
        nh2 = jnp.dot(pm2_ref[...], h2r_ref[...] * g2_ref[...],
                      preferred_element_type=jnp.float32)
        s = jax.nn.sigmoid(jnp.dot(nh2, g3w_ref[...]) + g3b_ref[0, 0])
        g = _softcol(s)
        gp3 = jnp.sum(g * nh2, axis=0, keepdims=True)
        return jnp.concatenate([gp1_ref[...], gp2_ref[...], gp3], axis=1)

    x1 = gvec(h2r1_ref, g21_ref, gp11_ref, gp21_ref)
    x2 = gvec(h2r2_ref, g22_ref, gp12_ref, gp22_ref)

    def cell(x, hprev, cprev, wx_ref, whh_ref, b_ref):
        z = jnp.dot(x, wx_ref[...], preferred_element_type=jnp.float32)
        z = z + b_ref[...]
        if hprev is not None:
            z = z + jnp.dot(hprev, whh_ref[...],
                            preferred_element_type=jnp.float32)
        i = z[:, 0:HID]
        f = z[:, HID:2 * HID]
        gg = z[:, 2 * HID:3 * HID]
        o = z[:, 3 * HID:4 * HID]
        ig = jax.nn.sigmoid(i) * jnp.tanh(gg)
        c = ig if cprev is None else jax.nn.sigmoid(f) * cprev + ig
        return jax.nn.sigmoid(o) * jnp.tanh(c), c

    h01, c0 = cell(x1, None, None, wx0_ref, wh0_ref, b0_ref)
    h02, _ = cell(x2, h01, c0, wx0_ref, wh0_ref, b0_ref)
    h11, c1 = cell(h01, None, None, wx1_ref, wh1_ref, b1_ref)
    h12, _ = cell(h02, h11, c1, wx1_ref, wh1_ref, b1_ref)
    z = jnp.dot(h12, wfc_ref[...]) + bfc_ref[...]
    mx = jnp.max(z, axis=1, keepdims=True)
    ez = jnp.exp(z - mx)
    out_ref[...] = ez / jnp.sum(ez, axis=1, keepdims=True)


def _sc_gate_call(p1s, p2s, ss, ds, n, shift):
    eps = E // NSUB
    sl = n // NSUB
    mesh = plsc.VectorSubcoreMesh(core_axis_name="c", subcore_axis_name="s")

    @functools.partial(
        pl.kernel,
        out_type=jax.ShapeDtypeStruct((2, n), jnp.float32),
        mesh=mesh,
        scratch_types=[
            pltpu.VMEM((n,), jnp.float32),
            pltpu.VMEM((n,), jnp.float32),
            pltpu.VMEM((eps,), jnp.int32),
            pltpu.VMEM((eps,), jnp.int32),
            pltpu.VMEM((n,), jnp.float32),
            pltpu.VMEM_SHARED((NSUB, n), jnp.float32),
            pltpu.VMEM((sl,), jnp.float32),
            pltpu.VMEM((sl,), jnp.float32),
            pltpu.SemaphoreType.DMA,
        ],
        compiler_params=_SC_PARAMS,
    )
    def k(p1_hbm, p2_hbm, s_hbm, d_hbm, out_hbm,
          p1_v, p2_v, s_v, d_v, gate_v, stage, acc, tmp, sem):
        c = lax.axis_index("c")
        w = lax.axis_index("s")
        pltpu.sync_copy(p1_hbm.at[c], p1_v)
        pltpu.sync_copy(p2_hbm.at[c], p2_v)
        pltpu.sync_copy(s_hbm.at[c, pl.ds(w * eps, eps)], s_v)
        pltpu.sync_copy(d_hbm.at[c, pl.ds(w * eps, eps)], d_v)

        @pl.loop(0, n, step=16)
        def _(i):
            gate_v[pl.ds(i, 16)] = jnp.full((16,), 0.5, jnp.float32)

        @pl.loop(0, eps, step=16)
        def _(i):
            s16 = s_v[pl.ds(i, 16)]
            d16 = d_v[pl.ds(i, 16)]
            if shift:
                s16 = jnp.right_shift(s16, shift)
                d16 = jnp.right_shift(d16, shift)
            a = plsc.load_gather(p1_v, [s16])
            b = plsc.load_gather(p2_v, [d16])
            sc = 1.0 / (1.0 + jnp.exp(-(a + b)))

            def cond(cur):
                return jnp.any(sc > cur)

            def body(cur):
                plsc.store_scatter(gate_v, [d16], sc, mask=sc > cur)
                return plsc.load_gather(gate_v, [d16])

            lax.while_loop(cond, body, plsc.load_gather(gate_v, [d16]))

        pltpu.sync_copy(gate_v, stage.at[w])
        plsc.subcore_barrier()
        pltpu.sync_copy(stage.at[0, pl.ds(w * sl, sl)], acc)
        for j in range(1, NSUB):
            pltpu.sync_copy(stage.at[j, pl.ds(w * sl, sl)], tmp)

            @pl.loop(0, sl, step=16)
            def _(v):
                acc[pl.ds(v, 16)] = jnp.maximum(acc[pl.ds(v, 16)],
                                                tmp[pl.ds(v, 16)])

        pltpu.sync_copy(acc, out_hbm.at[c, pl.ds(w * sl, sl)])

    return k(p1s, p2s, ss, ds)


def _sc_adj_call(ss, ds, bes):
    n2 = N // 2
    rows = n2 // NSUB
    ch = rows * n2
    mesh = plsc.VectorSubcoreMesh(core_axis_name="c", subcore_axis_name="s")

    big = 20000

    @functools.partial(
        pl.kernel,
        out_type=(jax.ShapeDtypeStruct((2, n2 * n2), jnp.float32),
                  jax.ShapeDtypeStruct((2, n2 * n2), jnp.float32)),
        mesh=mesh,
        scratch_types=[
            pltpu.VMEM((E,), jnp.int32),
            pltpu.VMEM((E,), jnp.int32),
            pltpu.VMEM((E,), jnp.float32),
            pltpu.VMEM((16,), jnp.int32),
            pltpu.VMEM((ch,), jnp.float32),
            pltpu.VMEM((ch,), jnp.float32),
            pltpu.SemaphoreType.DMA,
        ],
        compiler_params=_SC_PARAMS,
    )
    def k(s_hbm, d_hbm, be_hbm, b2_hbm, mf_hbm,
          s_v, d_v, be_v, nb_v, b2_v, mf_v, sem):
        c = lax.axis_index("c")
        w = lax.axis_index("s")
        pltpu.sync_copy(s_hbm.at[c], s_v)
        pltpu.sync_copy(d_hbm.at[c], d_v)
        pltpu.sync_copy(be_hbm.at[c], be_v)

        zf = jnp.zeros((16,), jnp.float32)

        @pl.loop(0, ch, step=16)
        def _(i):
            b2_v[pl.ds(i, 16)] = zf
            mf_v[pl.ds(i, 16)] = zf

        row0 = w * rows
        lane = lax.iota(jnp.int32, 16)
        ones = jnp.ones((16,), jnp.float32)

        @pl.loop(0, E, step=16, unroll=2)
        def _(i):
            s16 = jnp.right_shift(s_v[pl.ds(i, 16)], 1)
            d16 = jnp.right_shift(d_v[pl.ds(i, 16)], 1)
            srel = s16 - row0
            inr = (srel >= 0) & (srel < rows)
            lidx = jnp.where(inr, srel * n2 + d16, big)
            eid = lane + i
            ckey, eids = plsc.sort_key_val(lidx * 16384 + eid, eid)
            nb_v[...] = ckey
            nxt = plsc.load_gather(nb_v, [jnp.minimum(lane + 1, 15)])
            l2 = jnp.right_shift(ckey, 14)
            last = (jnp.right_shift(nxt, 14) != l2) | (lane == 15)
            ok = last & (ckey < (16384 * 16384))
            addr = jnp.minimum(l2, ch - 1)
            bg = plsc.load_gather(be_v, [eids])
            plsc.store_scatter(b2_v, [addr], bg, mask=ok)
            plsc.store_scatter(mf_v, [addr], ones, mask=ok)

        pltpu.sync_copy(b2_v, b2_hbm.at[c, pl.ds(w * ch, ch)])
        pltpu.sync_copy(mf_v, mf_hbm.at[c, pl.ds(w * ch, ch)])

    return k(ss, ds, bes)


_f32 = jnp.float32


def _p12_call(x, whw, g1w, g1b, wgcat, atop, abot, ea, wegf, ae2c):
    return pl.pallas_call(
        _p12_body,
        out_shape=(jax.ShapeDtypeStruct((1, HID), _f32),
                   jax.ShapeDtypeStruct((N, CAT), _f32),
                   jax.ShapeDtypeStruct((N, NH), _f32),
                   jax.ShapeDtypeStruct((NH, N), _f32),
                   jax.ShapeDtypeStruct((E, 1), _f32)),
    )(x, whw, g1w, g1b, wgcat, atop, abot, ea, wegf, ae2c)


def _a1_call(whcat, f1c, f2t, aeg, adj, nnp, u1, v1, bp1):
    bs = 128
    grid = (N // bs,)
    full2 = lambda a, b: pl.BlockSpec((a, b), lambda i: (0, 0))
    rows = lambda a, b: pl.BlockSpec((a, b), lambda i: (i, 0))
    return pl.pallas_call(
        _a1_body,
        grid=grid,
        in_specs=[full2(N, CAT), rows(bs, NH), full2(NH, N), full2(NH, DE),
                  rows(bs, N),
                  pl.BlockSpec((DE, bs, N), lambda i: (0, i, 0)),
                  full2(CAT, 1), full2(CAT, 1), full2(1, 1)],
        out_specs=[rows(bs, CAT), rows(bs, 1), rows(bs, 1)],
        out_shape=(jax.ShapeDtypeStruct((N, CAT), _f32),
                   jax.ShapeDtypeStruct((N, 1), _f32),
                   jax.ShapeDtypeStruct((N, 1), _f32)),
    )(whcat, f1c, f2t, aeg, adj, nnp, u1, v1, bp1)


def _pa_call(hcat, gcol, pm, g2w, g2b, w2, a2t, a2b, b2, mf, u2, v2, bp2):
    n2 = N // 2
    return pl.pallas_call(
        _pa_body,
        out_shape=(jax.ShapeDtypeStruct((1, CAT), _f32),
                   jax.ShapeDtypeStruct((n2, O1), _f32),
                   jax.ShapeDtypeStruct((n2, 1), _f32),
                   jax.ShapeDtypeStruct((n2, 1), _f32)),
    )(hcat, gcol, pm, g2w, g2b, w2, a2t, a2b, b2, mf, u2, v2, bp2)


def _f_call(h2r1, g21, h2r2, g22, pm2, g3w, g3b, gp11, gp21, gp12, gp22,
            wx0, wh0, b0, wx1, wh1, b1, wfc, bfc):
    return pl.pallas_call(
        _f_body,
        out_shape=jax.ShapeDtypeStruct((1, 2), _f32),
    )(h2r1, g21, h2r2, g22, pm2, g3w, g3b, gp11, gp21, gp12, gp22,
      wx0, wh0, b0, wx1, wh1, b1, wfc, bfc)


def kernel(features1, features2, edge_index1, edge_index2, edgesAttr1,
           edgesAttr2, adjacency1, adjacency2, node2node_features1,
           node2node_features2, params):
    p = params
    n2 = N // 2

    ei1 = edge_index1.astype(jnp.int32)
    ei2 = edge_index2.astype(jnp.int32)
    ss = jnp.stack([ei1[0], ei2[0]])
    ds = jnp.stack([ei1[1], ei2[1]])

    wgcat = jnp.transpose(p['Wg'], (1, 0, 2)).reshape(HID, CAT)
    atop = p['ag'][:, :O1]
    abot = p['ag'][:, O1:]
    wegf = jnp.transpose(p['Weg'], (1, 0, 2)).reshape(DE, NH * DE)
    ae2c = p['ae2'].reshape(NH * DE, 1)
    u1 = p['Wp1'][:CAT, :]
    v1 = p['Wp1'][CAT:, :]
    bp1 = p['bp1'].reshape(1, 1)
    a2t = p['a2'][None, :O1]
    a2b = p['a2'][None, O1:]
    u2 = p['Wp2'][:O1, :]
    v2 = p['Wp2'][O1:, :]
    bp2 = p['bp2'].reshape(1, 1)
    g1b = p['g1b'].reshape(1, 1)
    g2b = p['g2b'].reshape(1, 1)
    g3b = p['g3b'].reshape(1, 1)
    b0 = p['b0'][None]
    b1 = p['b1'][None]
    bfc = p['bfc'][None]

    def stage1(x, ea, nn, adj):
        gp1, whcat, f1c, f2t, be = _p12_call(
            x, p['W_h'], p['g1W'], g1b, wgcat, atop, abot, ea, wegf, ae2c)
        nnp = jnp.transpose(nn).reshape(DE, N, N)
        hcat, p1, p2 = _a1_call(whcat, f1c, f2t, p['aeg'], adj,
                                nnp, u1, v1, bp1)
        return gp1, hcat, p1, p2, be

    gp1_1, hcat1, p1_1, p2_1, be1 = stage1(
        features1, edgesAttr1, node2node_features1, adjacency1)
    gp1_2, hcat2, p1_2, p2_2, be2 = stage1(
        features2, edgesAttr2, node2node_features2, adjacency2)

    b2f, mff = _sc_adj_call(ss, ds, jnp.stack([be1[:, 0], be2[:, 0]]))
    gate1 = _sc_gate_call(jnp.stack([p1_1[:, 0], p1_2[:, 0]]),
                          jnp.stack([p2_1[:, 0], p2_2[:, 0]]),
                          ss, ds, N, 0)

    pm = jnp.repeat(jnp.eye(n2, dtype=_f32), 2, axis=1)
    pm2 = jnp.repeat(jnp.eye(n2 // 2, dtype=_f32), 2, axis=1)

    def stage2(hcat, g1row, b2row, mfrow):
        return _pa_call(
            hcat, g1row.reshape(N, 1), pm,
            p['g2W'], g2b, p['W2'], a2t, a2b,
            b2row.reshape(n2, n2), mfrow.reshape(n2, n2),
            u2, v2, bp2)

    gp2_1, h2_1, p1b_1, p2b_1 = stage2(hcat1, gate1[0], b2f[0], mff[0])
    gp2_2, h2_2, p1b_2, p2b_2 = stage2(hcat2, gate1[1], b2f[1], mff[1])

    gate2 = _sc_gate_call(jnp.stack([p1b_1[:, 0], p1b_2[:, 0]]),
                          jnp.stack([p2b_1[:, 0], p2b_2[:, 0]]),
                          ss, ds, n2, 1)

    return _f_call(h2_1, gate2[0].reshape(n2, 1),
                   h2_2, gate2[1].reshape(n2, 1), pm2,
                   p['g3W'], g3b, gp1_1, gp2_1, gp1_2, gp2_2,
                   p['Wx0'], p['Wh0'], b0, p['Wx1'], p['Wh1'], b1,
                   p['Wfc'], bfc)


# speedup vs baseline: 1.0630x; 1.0630x over previous
import dataclasses
import functools

import jax
import jax.numpy as jnp
from jax import lax
from jax.experimental import pallas as pl
from jax.experimental.pallas import tpu as pltpu
from jax.experimental.pallas import tpu_sc as plsc

N = 1024
E = 16384
HID = 128
DE = 4
NH = 16
ALPHA = 0.2
NEG = -1e9
O1 = 2 * HID
CAT = NH * O1
NSUB = 16

_SC_PARAMS = pltpu.CompilerParams()
if "needs_layout_passes" in pltpu.CompilerParams.__dataclass_fields__:
    _SC_PARAMS = dataclasses.replace(_SC_PARAMS, needs_layout_passes=False)


def _softcol(s):
    m = jnp.max(s, axis=0, keepdims=True)
    e = jnp.exp(s - m)
    return e * (1.0 / jnp.sum(e, axis=0, keepdims=True))


def _elu(x):
    return jnp.where(x > 0, x, jnp.exp(jnp.minimum(x, 0.0)) - 1.0)


def _p12_body(x_ref, whw_ref, g1w_ref, g1b_ref, wgcat_ref, atop_ref,
              abot_ref, ea_ref, wegf_ref, ae2_ref,
              gp1_ref, whcat_ref, f1_ref, f2t_ref, be_ref):
    h = jnp.dot(x_ref[...], whw_ref[...], preferred_element_type=jnp.float32)
    h = _elu(h)
    s = jax.nn.sigmoid(jnp.dot(h, g1w_ref[...]) + g1b_ref[0, 0])
    g = _softcol(s)
    gp1_ref[...] = jnp.sum(g * h, axis=0, keepdims=True)
    whcat = jnp.dot(h, wgcat_ref[...], preferred_element_type=jnp.float32)
    whcat_ref[...] = whcat
    for hd in range(NH):
        wh = whcat[:, O1 * hd:O1 * (hd + 1)]
        f1_ref[:, hd:hd + 1] = lax.dot_general(
            wh, atop_ref[hd:hd + 1, :], (((1,), (1,)), ((), ())))
        f2t_ref[hd:hd + 1, :] = lax.dot_general(
            abot_ref[hd:hd + 1, :], wh, (((1,), (1,)), ((), ())))
    wbe = jnp.dot(wegf_ref[...], ae2_ref[...])
    be_ref[...] = jnp.dot(ea_ref[...], wbe)


def _a1_body(whcat_ref, f1_ref, f2t_ref, aeg_ref, adj_ref,
             nn_ref, u1_ref, v1_ref, bp1_ref,
             hcat_ref, p1_ref, p2_ref):
    adjm = adj_ref[...] > 0
    nk = tuple(nn_ref[k] for k in range(DE))
    bs = f1_ref.shape[0]
    p1 = jnp.zeros((bs, 1), jnp.float32) + bp1_ref[0, 0]
    p2 = jnp.zeros((bs, 1), jnp.float32)
    for hd in range(NH):
        wh = whcat_ref[:, O1 * hd:O1 * (hd + 1)]
        bt = (nk[0] * aeg_ref[hd, 0] + nk[1] * aeg_ref[hd, 1]
              + nk[2] * aeg_ref[hd, 2] + nk[3] * aeg_ref[hd, 3])
        t = f1_ref[:, hd:hd + 1] + f2t_ref[hd:hd + 1, :] + bt
        t = jnp.where(t > 0, t, ALPHA * t)
        e = jnp.where(adjm, t, NEG)
        mx = jnp.max(e, axis=1, keepdims=True)
        ex = jnp.exp(e - mx)
        att = ex * (1.0 / jnp.sum(ex, axis=1, keepdims=True))
        hp = jnp.dot(att, wh, preferred_element_type=jnp.float32)
        hp = _elu(hp)
        hcat_ref[:, O1 * hd:O1 * (hd + 1)] = hp
        p1 = p1 + jnp.dot(hp, u1_ref[O1 * hd:O1 * (hd + 1), :])
        p2 = p2 + jnp.dot(hp, v1_ref[O1 * hd:O1 * (hd + 1), :])
    p1_ref[...] = p1
    p2_ref[...] = p2


def _pa_body(hcat_ref, gate_ref, pm_ref, g2w_ref, g2b_ref, w2_ref, a2t_ref,
             a2b_ref, b2_ref, mf_ref, u2_ref, v2_ref, bp2_ref,
             gp2_ref, h2_ref, p1b_ref, p2b_ref):
    nh = jnp.dot(pm_ref[...], hcat_ref[...] * gate_ref[...],
                 preferred_element_type=jnp.float32)
    s = jax.nn.sigmoid(jnp.dot(nh, g2w_ref[...]) + g2b_ref[0, 0])
    g = _softcol(s)
    gp2_ref[...] = jnp.sum(g * nh, axis=0, keepdims=True)
    wh2 = jnp.dot(nh, w2_ref[...], preferred_element_type=jnp.float32)
    f12 = lax.dot_general(wh2, a2t_ref[...], (((1,), (1,)), ((), ())))
    f2t2 = lax.dot_general(a2b_ref[...], wh2, (((1,), (1,)), ((), ())))
    t = f12 + f2t2 + b2_ref[...]
    t = jnp.where(t > 0, t, ALPHA * t)
    e = jnp.where(mf_ref[...] > 0, t, NEG)
    mx = jnp.max(e, axis=1, keepdims=True)
    ex = jnp.exp(e - mx)
    att = ex * (1.0 / jnp.sum(ex, axis=1, keepdims=True))
    h2 = jnp.dot(att, wh2, preferred_element_type=jnp.float32)
    h2_ref[...] = h2
    p1b_ref[...] = jnp.dot(h2, u2_ref[...]) + bp2_ref[0, 0]
    p2b_ref[...] = jnp.dot(h2, v2_ref[...])


def _f_body(h2r1_ref, g21_ref, h2r2_ref, g22_ref, pm2_ref, g3w_ref, g3b_ref,
            gp11_ref, gp21_ref, gp12_ref, gp22_ref,
            wx0_ref, wh0_ref, b0_ref, wx1_ref, wh1_ref, b1_ref,
            wfc_ref, bfc_ref, out_ref):
    def gvec(h2r_ref, g2_ref, gp1_ref, gp2_ref):
        nh2 = jnp.dot(pm2_ref[...], h2r_ref[...] * g2_ref[...],
                      preferred_element_type=jnp.float32)
        s = jax.nn.sigmoid(jnp.dot(nh2, g3w_ref[...]) + g3b_ref[0, 0])
        g = _softcol(s)
        gp3 = jnp.sum(g * nh2, axis=0, keepdims=True)
        return jnp.concatenate([gp1_ref[...], gp2_ref[...], gp3], axis=1)

    x1 = gvec(h2r1_ref, g21_ref, gp11_ref, gp21_ref)
    x2 = gvec(h2r2_ref, g22_ref, gp12_ref, gp22_ref)

    def cell(x, hprev, cprev, wx_ref, whh_ref, b_ref):
        z = jnp.dot(x, wx_ref[...], preferred_element_type=jnp.float32)
        z = z + b_ref[...]
        if hprev is not None:
            z = z + jnp.dot(hprev, whh_ref[...],
                            preferred_element_type=jnp.float32)
        i = z[:, 0:HID]
        f = z[:, HID:2 * HID]
        gg = z[:, 2 * HID:3 * HID]
        o = z[:, 3 * HID:4 * HID]
        ig = jax.nn.sigmoid(i) * jnp.tanh(gg)
        c = ig if cprev is None else jax.nn.sigmoid(f) * cprev + ig
        return jax.nn.sigmoid(o) * jnp.tanh(c), c

    h01, c0 = cell(x1, None, None, wx0_ref, wh0_ref, b0_ref)
    h02, _ = cell(x2, h01, c0, wx0_ref, wh0_ref, b0_ref)
    h11, c1 = cell(h01, None, None, wx1_ref, wh1_ref, b1_ref)
    h12, _ = cell(h02, h11, c1, wx1_ref, wh1_ref, b1_ref)
    z = jnp.dot(h12, wfc_ref[...]) + bfc_ref[...]
    mx = jnp.max(z, axis=1, keepdims=True)
    ez = jnp.exp(z - mx)
    out_ref[...] = ez / jnp.sum(ez, axis=1, keepdims=True)


def _sc_gate_call(p1s, p2s, ss, ds, n, shift):
    eps = E // NSUB
    sl = n // NSUB
    mesh = plsc.VectorSubcoreMesh(core_axis_name="c", subcore_axis_name="s")

    @functools.partial(
        pl.kernel,
        out_type=jax.ShapeDtypeStruct((2, n), jnp.float32),
        mesh=mesh,
        scratch_types=[
            pltpu.VMEM((n,), jnp.float32),
            pltpu.VMEM((n,), jnp.float32),
            pltpu.VMEM((eps,), jnp.int32),
            pltpu.VMEM((eps,), jnp.int32),
            pltpu.VMEM((n,), jnp.float32),
            pltpu.VMEM_SHARED((NSUB, n), jnp.float32),
            pltpu.VMEM((sl,), jnp.float32),
            pltpu.VMEM((sl,), jnp.float32),
            pltpu.SemaphoreType.DMA,
        ],
        compiler_params=_SC_PARAMS,
    )
    def k(p1_hbm, p2_hbm, s_hbm, d_hbm, out_hbm,
          p1_v, p2_v, s_v, d_v, gate_v, stage, acc, tmp, sem):
        c = lax.axis_index("c")
        w = lax.axis_index("s")
        pltpu.sync_copy(p1_hbm.at[c], p1_v)
        pltpu.sync_copy(p2_hbm.at[c], p2_v)
        pltpu.sync_copy(s_hbm.at[c, pl.ds(w * eps, eps)], s_v)
        pltpu.sync_copy(d_hbm.at[c, pl.ds(w * eps, eps)], d_v)

        @pl.loop(0, n, step=16)
        def _(i):
            gate_v[pl.ds(i, 16)] = jnp.full((16,), 0.5, jnp.float32)

        @pl.loop(0, eps, step=16)
        def _(i):
            s16 = s_v[pl.ds(i, 16)]
            d16 = d_v[pl.ds(i, 16)]
            if shift:
                s16 = jnp.right_shift(s16, shift)
                d16 = jnp.right_shift(d16, shift)
            a = plsc.load_gather(p1_v, [s16])
            b = plsc.load_gather(p2_v, [d16])
            sc = 1.0 / (1.0 + jnp.exp(-(a + b)))

            def cond(cur):
                return jnp.any(sc > cur)

            def body(cur):
                plsc.store_scatter(gate_v, [d16], sc, mask=sc > cur)
                return plsc.load_gather(gate_v, [d16])

            lax.while_loop(cond, body, plsc.load_gather(gate_v, [d16]))

        pltpu.sync_copy(gate_v, stage.at[w])
        plsc.subcore_barrier()
        pltpu.sync_copy(stage.at[0, pl.ds(w * sl, sl)], acc)
        for j in range(1, NSUB):
            pltpu.sync_copy(stage.at[j, pl.ds(w * sl, sl)], tmp)

            @pl.loop(0, sl, step=16)
            def _(v):
                acc[pl.ds(v, 16)] = jnp.maximum(acc[pl.ds(v, 16)],
                                                tmp[pl.ds(v, 16)])

        pltpu.sync_copy(acc, out_hbm.at[c, pl.ds(w * sl, sl)])

    return k(p1s, p2s, ss, ds)


def _sc_adj_call(ss, ds, bes):
    n2 = N // 2
    rows = n2 // NSUB
    ch = rows * n2
    mesh = plsc.VectorSubcoreMesh(core_axis_name="c", subcore_axis_name="s")

    big = 20000

    @functools.partial(
        pl.kernel,
        out_type=(jax.ShapeDtypeStruct((2, n2 * n2), jnp.float32),
                  jax.ShapeDtypeStruct((2, n2 * n2), jnp.float32)),
        mesh=mesh,
        scratch_types=[
            pltpu.VMEM((E,), jnp.int32),
            pltpu.VMEM((E,), jnp.int32),
            pltpu.VMEM((E,), jnp.float32),
            pltpu.VMEM((16,), jnp.int32),
            pltpu.VMEM((ch,), jnp.float32),
            pltpu.VMEM((ch,), jnp.float32),
            pltpu.SemaphoreType.DMA,
        ],
        compiler_params=_SC_PARAMS,
    )
    def k(s_hbm, d_hbm, be_hbm, b2_hbm, mf_hbm,
          s_v, d_v, be_v, nb_v, b2_v, mf_v, sem):
        c = lax.axis_index("c")
        w = lax.axis_index("s")
        pltpu.sync_copy(s_hbm.at[c], s_v)
        pltpu.sync_copy(d_hbm.at[c], d_v)
        pltpu.sync_copy(be_hbm.at[c], be_v)

        zf = jnp.zeros((16,), jnp.float32)

        @pl.loop(0, ch, step=16)
        def _(i):
            b2_v[pl.ds(i, 16)] = zf
            mf_v[pl.ds(i, 16)] = zf

        row0 = w * rows
        lane = lax.iota(jnp.int32, 16)
        ones = jnp.ones((16,), jnp.float32)

        @pl.loop(0, E, step=16, unroll=2)
        def _(i):
            s16 = jnp.right_shift(s_v[pl.ds(i, 16)], 1)
            d16 = jnp.right_shift(d_v[pl.ds(i, 16)], 1)
            srel = s16 - row0
            inr = (srel >= 0) & (srel < rows)
            lidx = jnp.where(inr, srel * n2 + d16, big)
            eid = lane + i
            ckey, eids = plsc.sort_key_val(lidx * 16384 + eid, eid)
            nb_v[...] = ckey
            nxt = plsc.load_gather(nb_v, [jnp.minimum(lane + 1, 15)])
            l2 = jnp.right_shift(ckey, 14)
            last = (jnp.right_shift(nxt, 14) != l2) | (lane == 15)
            ok = last & (ckey < (16384 * 16384))
            addr = jnp.minimum(l2, ch - 1)
            bg = plsc.load_gather(be_v, [eids])
            plsc.store_scatter(b2_v, [addr], bg, mask=ok)
            plsc.store_scatter(mf_v, [addr], ones, mask=ok)

        pltpu.sync_copy(b2_v, b2_hbm.at[c, pl.ds(w * ch, ch)])
        pltpu.sync_copy(mf_v, mf_hbm.at[c, pl.ds(w * ch, ch)])

    return k(ss, ds, bes)


_f32 = jnp.float32


def _p12_call(x, whw, g1w, g1b, wgcat, atop, abot, ea, wegf, ae2c):
    return pl.pallas_call(
        _p12_body,
        out_shape=(jax.ShapeDtypeStruct((1, HID), _f32),
                   jax.ShapeDtypeStruct((N, CAT), _f32),
                   jax.ShapeDtypeStruct((N, NH), _f32),
                   jax.ShapeDtypeStruct((NH, N), _f32),
                   jax.ShapeDtypeStruct((E, 1), _f32)),
    )(x, whw, g1w, g1b, wgcat, atop, abot, ea, wegf, ae2c)


def _a1_call(whcat, f1c, f2t, aeg, adj, nnp, u1, v1, bp1):
    bs = 256
    grid = (N // bs,)
    full2 = lambda a, b: pl.BlockSpec((a, b), lambda i: (0, 0))
    rows = lambda a, b: pl.BlockSpec((a, b), lambda i: (i, 0))
    return pl.pallas_call(
        _a1_body,
        grid=grid,
        in_specs=[full2(N, CAT), rows(bs, NH), full2(NH, N), full2(NH, DE),
                  rows(bs, N),
                  pl.BlockSpec((DE, bs, N), lambda i: (0, i, 0)),
                  full2(CAT, 1), full2(CAT, 1), full2(1, 1)],
        out_specs=[rows(bs, CAT), rows(bs, 1), rows(bs, 1)],
        out_shape=(jax.ShapeDtypeStruct((N, CAT), _f32),
                   jax.ShapeDtypeStruct((N, 1), _f32),
                   jax.ShapeDtypeStruct((N, 1), _f32)),
    )(whcat, f1c, f2t, aeg, adj, nnp, u1, v1, bp1)


def _pa_call(hcat, gcol, pm, g2w, g2b, w2, a2t, a2b, b2, mf, u2, v2, bp2):
    n2 = N // 2
    return pl.pallas_call(
        _pa_body,
        out_shape=(jax.ShapeDtypeStruct((1, CAT), _f32),
                   jax.ShapeDtypeStruct((n2, O1), _f32),
                   jax.ShapeDtypeStruct((n2, 1), _f32),
                   jax.ShapeDtypeStruct((n2, 1), _f32)),
    )(hcat, gcol, pm, g2w, g2b, w2, a2t, a2b, b2, mf, u2, v2, bp2)


def _f_call(h2r1, g21, h2r2, g22, pm2, g3w, g3b, gp11, gp21, gp12, gp22,
            wx0, wh0, b0, wx1, wh1, b1, wfc, bfc):
    return pl.pallas_call(
        _f_body,
        out_shape=jax.ShapeDtypeStruct((1, 2), _f32),
    )(h2r1, g21, h2r2, g22, pm2, g3w, g3b, gp11, gp21, gp12, gp22,
      wx0, wh0, b0, wx1, wh1, b1, wfc, bfc)


def kernel(features1, features2, edge_index1, edge_index2, edgesAttr1,
           edgesAttr2, adjacency1, adjacency2, node2node_features1,
           node2node_features2, params):
    p = params
    n2 = N // 2

    ei1 = edge_index1.astype(jnp.int32)
    ei2 = edge_index2.astype(jnp.int32)
    ss = jnp.stack([ei1[0], ei2[0]])
    ds = jnp.stack([ei1[1], ei2[1]])

    wgcat = jnp.transpose(p['Wg'], (1, 0, 2)).reshape(HID, CAT)
    atop = p['ag'][:, :O1]
    abot = p['ag'][:, O1:]
    wegf = jnp.transpose(p['Weg'], (1, 0, 2)).reshape(DE, NH * DE)
    ae2c = p['ae2'].reshape(NH * DE, 1)
    u1 = p['Wp1'][:CAT, :]
    v1 = p['Wp1'][CAT:, :]
    bp1 = p['bp1'].reshape(1, 1)
    a2t = p['a2'][None, :O1]
    a2b = p['a2'][None, O1:]
    u2 = p['Wp2'][:O1, :]
    v2 = p['Wp2'][O1:, :]
    bp2 = p['bp2'].reshape(1, 1)
    g1b = p['g1b'].reshape(1, 1)
    g2b = p['g2b'].reshape(1, 1)
    g3b = p['g3b'].reshape(1, 1)
    b0 = p['b0'][None]
    b1 = p['b1'][None]
    bfc = p['bfc'][None]

    def stage1(x, ea, nn, adj):
        gp1, whcat, f1c, f2t, be = _p12_call(
            x, p['W_h'], p['g1W'], g1b, wgcat, atop, abot, ea, wegf, ae2c)
        nnp = jnp.transpose(nn).reshape(DE, N, N)
        hcat, p1, p2 = _a1_call(whcat, f1c, f2t, p['aeg'], adj,
                                nnp, u1, v1, bp1)
        return gp1, hcat, p1, p2, be

    gp1_1, hcat1, p1_1, p2_1, be1 = stage1(
        features1, edgesAttr1, node2node_features1, adjacency1)
    gp1_2, hcat2, p1_2, p2_2, be2 = stage1(
        features2, edgesAttr2, node2node_features2, adjacency2)

    b2f, mff = _sc_adj_call(ss, ds, jnp.stack([be1[:, 0], be2[:, 0]]))
    gate1 = _sc_gate_call(jnp.stack([p1_1[:, 0], p1_2[:, 0]]),
                          jnp.stack([p2_1[:, 0], p2_2[:, 0]]),
                          ss, ds, N, 0)

    pm = jnp.repeat(jnp.eye(n2, dtype=_f32), 2, axis=1)
    pm2 = jnp.repeat(jnp.eye(n2 // 2, dtype=_f32), 2, axis=1)

    def stage2(hcat, g1row, b2row, mfrow):
        return _pa_call(
            hcat, g1row.reshape(N, 1), pm,
            p['g2W'], g2b, p['W2'], a2t, a2b,
            b2row.reshape(n2, n2), mfrow.reshape(n2, n2),
            u2, v2, bp2)

    gp2_1, h2_1, p1b_1, p2b_1 = stage2(hcat1, gate1[0], b2f[0], mff[0])
    gp2_2, h2_2, p1b_2, p2b_2 = stage2(hcat2, gate1[1], b2f[1], mff[1])

    gate2 = _sc_gate_call(jnp.stack([p1b_1[:, 0], p1b_2[:, 0]]),
                          jnp.stack([p2b_1[:, 0], p2b_2[:, 0]]),
                          ss, ds, n2, 1)

    return _f_call(h2_1, gate2[0].reshape(n2, 1),
                   h2_2, gate2[1].reshape(n2, 1), pm2,
                   p['g3W'], g3b, gp1_1, gp2_1, gp1_2, gp2_2,
                   p['Wx0'], p['Wh0'], b0, p['Wx1'], p['Wh1'], b1,
                   p['Wfc'], bfc)
